# baseline (speedup 1.0000x reference)
import functools

import jax
import jax.numpy as jnp
from jax import lax
from jax.experimental import pallas as pl
from jax.experimental.pallas import tpu as pltpu

NUM_EXPERTS = 8
TOP_K = 2
MODEL_DIM = 768
INTER_DIM = 768
T = 2048
N = T * TOP_K
BM = 128
NPAD = N + NUM_EXPERTS * BM
NWIN = NPAD // BM
_INTERPRET = False


def _routing_body(x_ref, gwt_ref, e1_ref, e2_ref, xb_ref, h1_ref, h2_ref):
    x = x_ref[...]
    logits = lax.dot_general(
        x, gwt_ref[...], (((1,), (0,)), ((), ())),
        preferred_element_type=jnp.float32)
    idx = lax.broadcasted_iota(jnp.int32, logits.shape, 1)
    r1 = jnp.max(logits, axis=1, keepdims=True)
    i1 = jnp.min(jnp.where(logits == r1, idx, NUM_EXPERTS),
                 axis=1, keepdims=True)
    l2 = jnp.where(idx == i1, -jnp.inf, logits)
    r2 = jnp.max(l2, axis=1, keepdims=True)
    i2 = jnp.min(jnp.where(l2 == r2, idx, NUM_EXPERTS),
                 axis=1, keepdims=True)
    e1_ref[...] = i1
    e2_ref[...] = i2
    lane16 = lax.broadcasted_iota(jnp.int32, (i1.shape[0], 16), 1)
    oh1 = (lane16 == i1).astype(jnp.int32)
    oh2 = (lane16 == i2).astype(jnp.int32)
    hb = i1.shape[0] // 2
    h1_ref[0, 0:1, :] = jnp.sum(oh1[:hb], axis=0, keepdims=True)
    h1_ref[0, 1:2, :] = jnp.sum(oh1[hb:], axis=0, keepdims=True)
    h2_ref[0, 0:1, :] = jnp.sum(oh2[:hb], axis=0, keepdims=True)
    h2_ref[0, 1:2, :] = jnp.sum(oh2[hb:], axis=0, keepdims=True)
    half = MODEL_DIM // 2
    lo = pltpu.bitcast(x[:, :half].astype(jnp.bfloat16).astype(jnp.float32),
                       jnp.int32)
    hi = pltpu.bitcast(x[:, half:].astype(jnp.bfloat16).astype(jnp.float32),
                       jnp.int32)
    xb_ref[...] = hi | lax.shift_right_logical(lo, 16)


def _routing(x, gwt):
    bt = 256
    return pl.pallas_call(
        _routing_body,
        grid=(T // bt,),
        in_specs=[
            pl.BlockSpec((bt, MODEL_DIM), lambda i: (i, 0)),
            pl.BlockSpec((MODEL_DIM, NUM_EXPERTS), lambda i: (0, 0)),
        ],
        out_specs=[
            pl.BlockSpec((bt, 1), lambda i: (i, 0)),
            pl.BlockSpec((bt, 1), lambda i: (i, 0)),
            pl.BlockSpec((bt, MODEL_DIM // 2), lambda i: (i, 0)),
            pl.BlockSpec((1, 2, 16), lambda i: (i, 0, 0)),
            pl.BlockSpec((1, 2, 16), lambda i: (i, 0, 0)),
        ],
        out_shape=[
            jax.ShapeDtypeStruct((T, 1), jnp.int32),
            jax.ShapeDtypeStruct((T, 1), jnp.int32),
            jax.ShapeDtypeStruct((T, MODEL_DIM // 2), jnp.int32),
            jax.ShapeDtypeStruct((T // 256, 2, 16), jnp.int32),
            jax.ShapeDtypeStruct((T // 256, 2, 16), jnp.int32),
        ],
        interpret=_INTERPRET,
    )(x, gwt)


NC = 2
NS = 16
LANES = 16
NW = NC * NS
SLOTS_W = N // NW
CHUNK = 32
NCHUNK = SLOTS_W // CHUNK
TOK_W = T // NW
NWEXP = 48


def _sc_mesh():
    from jax.experimental.pallas import tpu_sc as plsc
    return plsc.VectorSubcoreMesh(core_axis_name="c", subcore_axis_name="s")


def _dispatch_body(eids_hbm, x_hbm, h1_hbm, h2_hbm,
                   xs_hbm, inv_hbm, wexp_hbm, ps_hbm, cnt_hbm,
                   eids_v, hist_v, pos2d, inv_v, meta_v, wexp_v,
                   bufa, bufb, sema, semb):
    from jax.experimental.pallas import tpu_sc as plsc
    wid = lax.axis_index("c") * NS + lax.axis_index("s")
    base = wid * SLOTS_W
    lane = lax.iota(jnp.int32, LANES)

    pltpu.sync_copy(eids_hbm.at[pl.ds(base, SLOTS_W)], eids_v)
    pltpu.sync_copy(h1_hbm, hist_v.at[pl.ds(0, NW // 2)])
    pltpu.sync_copy(h2_hbm, hist_v.at[pl.ds(NW // 2, NW // 2)])

    tot = jnp.zeros((LANES,), jnp.int32)
    pre = jnp.zeros((LANES,), jnp.int32)
    for r in range(NW):
        row = hist_v[r, :]
        tot = tot + row
        pre = pre + jnp.where(r < wid, row, 0)

    pe = jnp.bitwise_and(tot + (BM - 1), -BM)
    incl = plsc.cumsum(pe)
    ps = incl - pe
    myb = ps + pre

    @pl.when(wid == 0)
    def _():
        meta_v[...] = ps
        pltpu.sync_copy(meta_v, ps_hbm)
        meta_v[...] = tot
        pltpu.sync_copy(meta_v, cnt_hbm)
        for k in range(NWEXP // LANES):
            wstart = (lane + k * LANES) * BM
            acc = jnp.zeros((LANES,), jnp.int32)
            for e in range(NUM_EXPERTS):
                incl_e = jnp.sum(jnp.where(lane == e, incl, 0))
                acc = acc + (wstart >= incl_e).astype(jnp.int32)
            wexp_v[pl.ds(k * LANES, LANES)] = jnp.minimum(acc, NUM_EXPERTS - 1)
        pltpu.sync_copy(wexp_v, wexp_hbm)

    for k in range(SLOTS_W // LANES):
        v = eids_v[pl.ds(k * LANES, LANES)]
        pos = jnp.zeros((LANES,), jnp.int32)
        for e in range(NUM_EXPERTS):
            m = v == e
            mi = m.astype(jnp.int32)
            r = plsc.cumsum(mi)
            base_e = jnp.sum(jnp.where(lane == e, myb, 0))
            pos = jnp.where(m, base_e + r - 1, pos)
            cnt_e = jnp.sum(mi)
            myb = myb + jnp.where(lane == e, cnt_e, 0)
        pos2d[k // 2, pl.ds((k % 2) * LANES, LANES)] = pos
        inv_v[pl.ds(k * LANES, LANES)] = pos
    pltpu.sync_copy(inv_v, inv_hbm.at[pl.ds(base, SLOTS_W)])

    handles = [None] * NCHUNK
    for c in range(NCHUNK):
        buf, sem = (bufa, sema) if c % 2 == 0 else (bufb, semb)
        if c >= 2:
            handles[c - 2].wait()
        tokbase = lax.rem(base + c * CHUNK, T)
        pltpu.sync_copy(x_hbm.at[pl.ds(tokbase, CHUNK)], buf)
        handles[c] = pltpu.async_copy(buf, xs_hbm.at[pos2d.at[c]], sem)
    handles[NCHUNK - 2].wait()
    handles[NCHUNK - 1].wait()


def _dispatch_sc(eids, x, h1, h2):
    f = functools.partial(
        pl.kernel,
        out_type=[
            jax.ShapeDtypeStruct((NPAD, MODEL_DIM // 2), jnp.int32),
            jax.ShapeDtypeStruct((N,), jnp.int32),
            jax.ShapeDtypeStruct((NWEXP,), jnp.int32),
            jax.ShapeDtypeStruct((LANES,), jnp.int32),
            jax.ShapeDtypeStruct((LANES,), jnp.int32),
        ],
        mesh=_sc_mesh(),
        compiler_params=pltpu.CompilerParams(needs_layout_passes=False),
        scratch_types=[
            pltpu.VMEM((SLOTS_W,), jnp.int32),
            pltpu.VMEM((NW, 16), jnp.int32),
            pltpu.VMEM((NCHUNK, CHUNK), jnp.int32),
            pltpu.VMEM((SLOTS_W,), jnp.int32),
            pltpu.VMEM((LANES,), jnp.int32),
            pltpu.VMEM((NWEXP,), jnp.int32),
            pltpu.VMEM((CHUNK, MODEL_DIM // 2), jnp.int32),
            pltpu.VMEM((CHUNK, MODEL_DIM // 2), jnp.int32),
            pltpu.SemaphoreType.DMA,
            pltpu.SemaphoreType.DMA,
        ],
    )(_dispatch_body)
    return f(eids, x, h1, h2)


def _combine_body(rows_hbm, inv_hbm, out_hbm, idx_v, r1, r2, obf, s1, s2):
    from jax.experimental.pallas import tpu_sc as plsc
    wid = lax.axis_index("c") * NS + lax.axis_index("s")
    tb = wid * TOK_W
    half = MODEL_DIM // 2
    lane = lax.iota(jnp.int32, LANES)
    pltpu.sync_copy(inv_hbm.at[pl.ds(tb, CHUNK)], idx_v.at[0])
    pltpu.sync_copy(inv_hbm.at[pl.ds(tb + CHUNK, CHUNK)], idx_v.at[1])
    pltpu.sync_copy(inv_hbm.at[pl.ds(T + tb, CHUNK)], idx_v.at[2])
    pltpu.sync_copy(inv_hbm.at[pl.ds(T + tb + CHUNK, CHUNK)], idx_v.at[3])
    groups_row = half // LANES
    for c in range(TOK_W // CHUNK):
        h1 = pltpu.async_copy(rows_hbm.at[idx_v.at[c]], r1, s1)
        h2 = pltpu.async_copy(rows_hbm.at[idx_v.at[2 + c]], r2, s2)
        h1.wait()
        h2.wait()

        def add_row(i, _):
            off = i * MODEL_DIM
            for j in range(groups_row):
                sl = pl.ds(j * LANES, LANES)
                w1 = r1[i, sl]
                w2 = r2[i, sl]
                losum = (plsc.bitcast(w1 << 16, jnp.float32)
                         + plsc.bitcast(w2 << 16, jnp.float32))
                hisum = (plsc.bitcast(w1 & -65536, jnp.float32)
                         + plsc.bitcast(w2 & -65536, jnp.float32))
                obf[pl.ds(off + j * LANES, LANES)] = losum
                obf[pl.ds(off + half + j * LANES, LANES)] = hisum
            return 0

        lax.fori_loop(0, CHUNK, add_row, 0)
        pltpu.sync_copy(obf, out_hbm.at[pl.ds((tb + c * CHUNK) * MODEL_DIM,
                                              CHUNK * MODEL_DIM)])


def _combine_sc(rows, inv):
    f = functools.partial(
        pl.kernel,
        out_type=jax.ShapeDtypeStruct((T * MODEL_DIM,), jnp.float32),
        mesh=_sc_mesh(),
        compiler_params=pltpu.CompilerParams(needs_layout_passes=False),
        scratch_types=[
            pltpu.VMEM((4, CHUNK), jnp.int32),
            pltpu.VMEM((CHUNK, MODEL_DIM // 2), jnp.int32),
            pltpu.VMEM((CHUNK, MODEL_DIM // 2), jnp.int32),
            pltpu.VMEM((CHUNK * MODEL_DIM,), jnp.float32),
            pltpu.SemaphoreType.DMA,
            pltpu.SemaphoreType.DMA,
        ],
    )(_combine_body)
    return f(rows, inv)


def _dispatch_jnp(x, eids):
    counts = jnp.bincount(eids, length=NUM_EXPERTS)
    padded = (counts + BM - 1) // BM * BM
    pad_start = jnp.concatenate([jnp.zeros((1,), jnp.int32),
                                 jnp.cumsum(padded)[:-1].astype(jnp.int32)])
    cs_excl = jnp.concatenate([jnp.zeros((1,), jnp.int32),
                               jnp.cumsum(counts)[:-1].astype(jnp.int32)])
    order = jnp.argsort(eids, stable=True)
    eids_sorted = eids[order]
    rank = jnp.arange(N, dtype=jnp.int32) - cs_excl[eids_sorted]
    pos_sorted = pad_start[eids_sorted] + rank
    inv = jnp.zeros((N,), jnp.int32).at[order].set(pos_sorted)
    tok = jnp.arange(N, dtype=jnp.int32) % T
    x_sorted = jnp.zeros((NPAD, MODEL_DIM), x.dtype).at[inv].set(x[tok])
    pad_incl = pad_start + padded
    wstarts = jnp.arange(NWIN, dtype=jnp.int32) * BM
    wexp = jnp.minimum(
        jnp.sum(wstarts[:, None] >= pad_incl[None, :], axis=1),
        NUM_EXPERTS - 1).astype(jnp.int32)
    return x_sorted, inv, wexp, pad_start.astype(jnp.int32), counts.astype(jnp.int32)


def _gemm_body(wexp_ref, ps_ref, cnt_ref, xs_ref, u_ref, d_ref, out_ref):
    w = pl.program_id(0)
    e = wexp_ref[w]
    loc = (lax.broadcasted_iota(jnp.int32, (BM, 1), 0)
           + w * BM - ps_ref[e])
    active = loc < cnt_ref[e]
    packed = jnp.where(active, xs_ref[...], 0)
    lo = pltpu.bitcast(lax.shift_left(packed, 16), jnp.float32)
    hi = pltpu.bitcast(packed & -65536, jnp.float32)
    x = jnp.concatenate([lo, hi], axis=1).astype(jnp.bfloat16)
    u = lax.dot_general(x, u_ref[0], (((1,), (0,)), ((), ())),
                        preferred_element_type=jnp.float32)
    h = (u * u).astype(jnp.bfloat16)
    acc = lax.dot_general(h, d_ref[0], (((1,), (0,)), ((), ())),
                          preferred_element_type=jnp.float32)
    half = MODEL_DIM // 2
    alo = pltpu.bitcast(acc[:, :half].astype(jnp.bfloat16).astype(jnp.float32),
                        jnp.int32)
    ahi = pltpu.bitcast(acc[:, half:].astype(jnp.bfloat16).astype(jnp.float32),
                        jnp.int32)
    out_ref[...] = ahi | lax.shift_right_logical(alo, 16)


def _grouped_gemm(x_sorted, u, d, wexp, pad_start, counts):
    grid_spec = pltpu.PrefetchScalarGridSpec(
        num_scalar_prefetch=3,
        grid=(NWIN,),
        in_specs=[
            pl.BlockSpec((BM, MODEL_DIM // 2), lambda i, wexp, ps, cnt: (i, 0)),
            pl.BlockSpec((1, MODEL_DIM, INTER_DIM),
                         lambda i, wexp, ps, cnt: (wexp[i], 0, 0)),
            pl.BlockSpec((1, INTER_DIM, MODEL_DIM),
                         lambda i, wexp, ps, cnt: (wexp[i], 0, 0)),
        ],
        out_specs=pl.BlockSpec((BM, MODEL_DIM // 2),
                               lambda i, wexp, ps, cnt: (i, 0)),
    )
    return pl.pallas_call(
        _gemm_body,
        grid_spec=grid_spec,
        out_shape=jax.ShapeDtypeStruct((NPAD, MODEL_DIM // 2), jnp.int32),
        interpret=_INTERPRET,
    )(wexp, pad_start, counts, x_sorted, u, d)


@functools.partial(jax.jit, static_argnames=())
def kernel(hidden_states, gate_w, gate_up_w, down_w):
    orig_shape = hidden_states.shape
    x = hidden_states.reshape(-1, orig_shape[-1])
    u = gate_up_w[:, :, INTER_DIM:].astype(jnp.bfloat16)
    d = down_w.astype(jnp.bfloat16)

    e1, e2, xb, h1, h2 = _routing(x, gate_w.T)
    eids = jnp.concatenate([e1, e2], axis=0).reshape(-1)

    x_sorted, inv, wexp, pad_start, counts = _dispatch_sc(
        eids, xb, h1.reshape(NW // 2, 16), h2.reshape(NW // 2, 16))
    rows = _grouped_gemm(x_sorted, u, d, wexp, pad_start, counts)
    out = _combine_sc(rows, inv)
    return out.reshape(orig_shape)


# speedup vs baseline: 1.0803x; 1.0803x over previous
import functools

import jax
import jax.numpy as jnp
from jax import lax
from jax.experimental import pallas as pl
from jax.experimental.pallas import tpu as pltpu

NUM_EXPERTS = 8
TOP_K = 2
MODEL_DIM = 768
INTER_DIM = 768
T = 2048
N = T * TOP_K
BM = 256
NPAD = N + NUM_EXPERTS * BM
NWIN = NPAD // BM
_INTERPRET = False


def _routing_body(x_ref, gwt_ref, e1_ref, e2_ref, xb_ref, h1_ref, h2_ref):
    x = x_ref[...]
    logits = lax.dot_general(
        x, gwt_ref[...], (((1,), (0,)), ((), ())),
        preferred_element_type=jnp.float32)
    idx = lax.broadcasted_iota(jnp.int32, logits.shape, 1)
    r1 = jnp.max(logits, axis=1, keepdims=True)
    i1 = jnp.min(jnp.where(logits == r1, idx, NUM_EXPERTS),
                 axis=1, keepdims=True)
    l2 = jnp.where(idx == i1, -jnp.inf, logits)
    r2 = jnp.max(l2, axis=1, keepdims=True)
    i2 = jnp.min(jnp.where(l2 == r2, idx, NUM_EXPERTS),
                 axis=1, keepdims=True)
    e1_ref[...] = i1
    e2_ref[...] = i2
    lane16 = lax.broadcasted_iota(jnp.int32, (i1.shape[0], 16), 1)
    oh1 = (lane16 == i1).astype(jnp.int32)
    oh2 = (lane16 == i2).astype(jnp.int32)
    hb = i1.shape[0] // 2
    h1_ref[0, 0:1, :] = jnp.sum(oh1[:hb], axis=0, keepdims=True)
    h1_ref[0, 1:2, :] = jnp.sum(oh1[hb:], axis=0, keepdims=True)
    h2_ref[0, 0:1, :] = jnp.sum(oh2[:hb], axis=0, keepdims=True)
    h2_ref[0, 1:2, :] = jnp.sum(oh2[hb:], axis=0, keepdims=True)
    half = MODEL_DIM // 2
    lo = pltpu.bitcast(x[:, :half].astype(jnp.bfloat16).astype(jnp.float32),
                       jnp.int32)
    hi = pltpu.bitcast(x[:, half:].astype(jnp.bfloat16).astype(jnp.float32),
                       jnp.int32)
    xb_ref[...] = hi | lax.shift_right_logical(lo, 16)


def _routing(x, gwt):
    bt = 256
    return pl.pallas_call(
        _routing_body,
        grid=(T // bt,),
        in_specs=[
            pl.BlockSpec((bt, MODEL_DIM), lambda i: (i, 0)),
            pl.BlockSpec((MODEL_DIM, NUM_EXPERTS), lambda i: (0, 0)),
        ],
        out_specs=[
            pl.BlockSpec((bt, 1), lambda i: (i, 0)),
            pl.BlockSpec((bt, 1), lambda i: (i, 0)),
            pl.BlockSpec((bt, MODEL_DIM // 2), lambda i: (i, 0)),
            pl.BlockSpec((1, 2, 16), lambda i: (i, 0, 0)),
            pl.BlockSpec((1, 2, 16), lambda i: (i, 0, 0)),
        ],
        out_shape=[
            jax.ShapeDtypeStruct((T, 1), jnp.int32),
            jax.ShapeDtypeStruct((T, 1), jnp.int32),
            jax.ShapeDtypeStruct((T, MODEL_DIM // 2), jnp.int32),
            jax.ShapeDtypeStruct((T // 256, 2, 16), jnp.int32),
            jax.ShapeDtypeStruct((T // 256, 2, 16), jnp.int32),
        ],
        interpret=_INTERPRET,
    )(x, gwt)


NC = 2
NS = 16
LANES = 16
NW = NC * NS
SLOTS_W = N // NW
CHUNK = 32
NCHUNK = SLOTS_W // CHUNK
TOK_W = T // NW
NWEXP = 48


def _sc_mesh():
    from jax.experimental.pallas import tpu_sc as plsc
    return plsc.VectorSubcoreMesh(core_axis_name="c", subcore_axis_name="s")


def _dispatch_body(e1_hbm, e2_hbm, x_hbm, h1_hbm, h2_hbm,
                   xs_hbm, inv_hbm, wexp_hbm, ps_hbm, cnt_hbm,
                   eids_v, hist_v, pos2d, inv_v, meta_v, wexp_v,
                   bufa, bufb, sema, semb):
    from jax.experimental.pallas import tpu_sc as plsc
    wid = lax.axis_index("c") * NS + lax.axis_index("s")
    base = wid * SLOTS_W
    lane = lax.iota(jnp.int32, LANES)

    @pl.when(wid < NW // 2)
    def _():
        pltpu.sync_copy(e1_hbm.at[pl.ds(wid * SLOTS_W, SLOTS_W)], eids_v)

    @pl.when(wid >= NW // 2)
    def _():
        pltpu.sync_copy(e2_hbm.at[pl.ds((wid - NW // 2) * SLOTS_W, SLOTS_W)],
                        eids_v)
    pltpu.sync_copy(h1_hbm, hist_v.at[pl.ds(0, NW // 2)])
    pltpu.sync_copy(h2_hbm, hist_v.at[pl.ds(NW // 2, NW // 2)])

    tot = jnp.zeros((LANES,), jnp.int32)
    pre = jnp.zeros((LANES,), jnp.int32)
    for r in range(NW):
        row = hist_v[r, :]
        tot = tot + row
        pre = pre + jnp.where(r < wid, row, 0)

    pe = jnp.bitwise_and(tot + (BM - 1), -BM)
    incl = plsc.cumsum(pe)
    ps = incl - pe
    myb = ps + pre

    @pl.when(wid == 0)
    def _():
        meta_v[...] = ps
        pltpu.sync_copy(meta_v, ps_hbm)
        meta_v[...] = tot
        pltpu.sync_copy(meta_v, cnt_hbm)
        for k in range(NWEXP // LANES):
            wstart = (lane + k * LANES) * BM
            acc = jnp.zeros((LANES,), jnp.int32)
            for e in range(NUM_EXPERTS):
                incl_e = jnp.sum(jnp.where(lane == e, incl, 0))
                acc = acc + (wstart >= incl_e).astype(jnp.int32)
            wexp_v[pl.ds(k * LANES, LANES)] = jnp.minimum(acc, NUM_EXPERTS - 1)
        pltpu.sync_copy(wexp_v, wexp_hbm)

    for k in range(SLOTS_W // LANES):
        v = eids_v[pl.ds(k * LANES, LANES)]
        pos = jnp.zeros((LANES,), jnp.int32)
        for e in range(NUM_EXPERTS):
            m = v == e
            mi = m.astype(jnp.int32)
            r = plsc.cumsum(mi)
            base_e = jnp.sum(jnp.where(lane == e, myb, 0))
            pos = jnp.where(m, base_e + r - 1, pos)
            cnt_e = jnp.sum(mi)
            myb = myb + jnp.where(lane == e, cnt_e, 0)
        pos2d[k // 2, pl.ds((k % 2) * LANES, LANES)] = pos
        inv_v[pl.ds(k * LANES, LANES)] = pos
    pltpu.sync_copy(inv_v, inv_hbm.at[pl.ds(base, SLOTS_W)])

    handles = [None] * NCHUNK
    for c in range(NCHUNK):
        buf, sem = (bufa, sema) if c % 2 == 0 else (bufb, semb)
        if c >= 2:
            handles[c - 2].wait()
        tokbase = lax.rem(base + c * CHUNK, T)
        pltpu.sync_copy(x_hbm.at[pl.ds(tokbase, CHUNK)], buf)
        handles[c] = pltpu.async_copy(buf, xs_hbm.at[pos2d.at[c]], sem)
    handles[NCHUNK - 2].wait()
    handles[NCHUNK - 1].wait()


def _dispatch_sc(e1, e2, x, h1, h2):
    f = functools.partial(
        pl.kernel,
        out_type=[
            jax.ShapeDtypeStruct((NPAD, MODEL_DIM // 2), jnp.int32),
            jax.ShapeDtypeStruct((N,), jnp.int32),
            jax.ShapeDtypeStruct((NWEXP,), jnp.int32),
            jax.ShapeDtypeStruct((LANES,), jnp.int32),
            jax.ShapeDtypeStruct((LANES,), jnp.int32),
        ],
        mesh=_sc_mesh(),
        compiler_params=pltpu.CompilerParams(needs_layout_passes=False),
        scratch_types=[
            pltpu.VMEM((SLOTS_W,), jnp.int32),
            pltpu.VMEM((NW, 16), jnp.int32),
            pltpu.VMEM((NCHUNK, CHUNK), jnp.int32),
            pltpu.VMEM((SLOTS_W,), jnp.int32),
            pltpu.VMEM((LANES,), jnp.int32),
            pltpu.VMEM((NWEXP,), jnp.int32),
            pltpu.VMEM((CHUNK, MODEL_DIM // 2), jnp.int32),
            pltpu.VMEM((CHUNK, MODEL_DIM // 2), jnp.int32),
            pltpu.SemaphoreType.DMA,
            pltpu.SemaphoreType.DMA,
        ],
    )(_dispatch_body)
    return f(e1, e2, x, h1, h2)


def _combine_body(rows_hbm, inv_hbm, out_hbm, idx_v, r1, r2, obf, s1, s2):
    from jax.experimental.pallas import tpu_sc as plsc
    wid = lax.axis_index("c") * NS + lax.axis_index("s")
    tb = wid * TOK_W
    half = MODEL_DIM // 2
    lane = lax.iota(jnp.int32, LANES)
    pltpu.sync_copy(inv_hbm.at[pl.ds(tb, CHUNK)], idx_v.at[0])
    pltpu.sync_copy(inv_hbm.at[pl.ds(tb + CHUNK, CHUNK)], idx_v.at[1])
    pltpu.sync_copy(inv_hbm.at[pl.ds(T + tb, CHUNK)], idx_v.at[2])
    pltpu.sync_copy(inv_hbm.at[pl.ds(T + tb + CHUNK, CHUNK)], idx_v.at[3])
    groups_row = half // LANES
    for c in range(TOK_W // CHUNK):
        h1 = pltpu.async_copy(rows_hbm.at[idx_v.at[c]], r1, s1)
        h2 = pltpu.async_copy(rows_hbm.at[idx_v.at[2 + c]], r2, s2)
        h1.wait()
        h2.wait()

        def add_row(i, _):
            off = i * MODEL_DIM
            for j in range(groups_row):
                sl = pl.ds(j * LANES, LANES)
                w1 = r1[i, sl]
                w2 = r2[i, sl]
                losum = (plsc.bitcast(w1 << 16, jnp.float32)
                         + plsc.bitcast(w2 << 16, jnp.float32))
                hisum = (plsc.bitcast(w1 & -65536, jnp.float32)
                         + plsc.bitcast(w2 & -65536, jnp.float32))
                obf[pl.ds(off + j * LANES, LANES)] = losum
                obf[pl.ds(off + half + j * LANES, LANES)] = hisum
            return 0

        lax.fori_loop(0, CHUNK, add_row, 0)
        pltpu.sync_copy(obf, out_hbm.at[pl.ds((tb + c * CHUNK) * MODEL_DIM,
                                              CHUNK * MODEL_DIM)])


def _combine_sc(rows, inv):
    f = functools.partial(
        pl.kernel,
        out_type=jax.ShapeDtypeStruct((T * MODEL_DIM,), jnp.float32),
        mesh=_sc_mesh(),
        compiler_params=pltpu.CompilerParams(needs_layout_passes=False),
        scratch_types=[
            pltpu.VMEM((4, CHUNK), jnp.int32),
            pltpu.VMEM((CHUNK, MODEL_DIM // 2), jnp.int32),
            pltpu.VMEM((CHUNK, MODEL_DIM // 2), jnp.int32),
            pltpu.VMEM((CHUNK * MODEL_DIM,), jnp.float32),
            pltpu.SemaphoreType.DMA,
            pltpu.SemaphoreType.DMA,
        ],
    )(_combine_body)
    return f(rows, inv)


def _dispatch_jnp(x, eids):
    counts = jnp.bincount(eids, length=NUM_EXPERTS)
    padded = (counts + BM - 1) // BM * BM
    pad_start = jnp.concatenate([jnp.zeros((1,), jnp.int32),
                                 jnp.cumsum(padded)[:-1].astype(jnp.int32)])
    cs_excl = jnp.concatenate([jnp.zeros((1,), jnp.int32),
                               jnp.cumsum(counts)[:-1].astype(jnp.int32)])
    order = jnp.argsort(eids, stable=True)
    eids_sorted = eids[order]
    rank = jnp.arange(N, dtype=jnp.int32) - cs_excl[eids_sorted]
    pos_sorted = pad_start[eids_sorted] + rank
    inv = jnp.zeros((N,), jnp.int32).at[order].set(pos_sorted)
    tok = jnp.arange(N, dtype=jnp.int32) % T
    x_sorted = jnp.zeros((NPAD, MODEL_DIM), x.dtype).at[inv].set(x[tok])
    pad_incl = pad_start + padded
    wstarts = jnp.arange(NWIN, dtype=jnp.int32) * BM
    wexp = jnp.minimum(
        jnp.sum(wstarts[:, None] >= pad_incl[None, :], axis=1),
        NUM_EXPERTS - 1).astype(jnp.int32)
    return x_sorted, inv, wexp, pad_start.astype(jnp.int32), counts.astype(jnp.int32)


def _gemm_body(wexp_ref, ps_ref, cnt_ref, xs_ref, u_ref, d_ref, out_ref):
    w = pl.program_id(0)
    e = wexp_ref[w]
    loc = (lax.broadcasted_iota(jnp.int32, (BM, 1), 0)
           + w * BM - ps_ref[e])
    active = loc < cnt_ref[e]

    @pl.when(w * BM - ps_ref[e] < cnt_ref[e])
    def _():
        _gemm_compute(active, xs_ref, u_ref, d_ref, out_ref)


def _gemm_compute(active, xs_ref, u_ref, d_ref, out_ref):
    packed = jnp.where(active, xs_ref[...], 0)
    lo = pltpu.bitcast(lax.shift_left(packed, 16), jnp.float32)
    hi = pltpu.bitcast(packed & -65536, jnp.float32)
    x = jnp.concatenate([lo, hi], axis=1).astype(jnp.bfloat16)
    u = lax.dot_general(x, u_ref[0], (((1,), (0,)), ((), ())),
                        preferred_element_type=jnp.float32)
    h = (u * u).astype(jnp.bfloat16)
    acc = lax.dot_general(h, d_ref[0], (((1,), (0,)), ((), ())),
                          preferred_element_type=jnp.float32)
    half = MODEL_DIM // 2
    alo = pltpu.bitcast(acc[:, :half].astype(jnp.bfloat16).astype(jnp.float32),
                        jnp.int32)
    ahi = pltpu.bitcast(acc[:, half:].astype(jnp.bfloat16).astype(jnp.float32),
                        jnp.int32)
    out_ref[...] = ahi | lax.shift_right_logical(alo, 16)


def _grouped_gemm(x_sorted, u, d, wexp, pad_start, counts):
    grid_spec = pltpu.PrefetchScalarGridSpec(
        num_scalar_prefetch=3,
        grid=(NWIN,),
        in_specs=[
            pl.BlockSpec((BM, MODEL_DIM // 2), lambda i, wexp, ps, cnt: (i, 0)),
            pl.BlockSpec((1, MODEL_DIM, INTER_DIM),
                         lambda i, wexp, ps, cnt: (wexp[i], 0, 0)),
            pl.BlockSpec((1, INTER_DIM, MODEL_DIM),
                         lambda i, wexp, ps, cnt: (wexp[i], 0, 0)),
        ],
        out_specs=pl.BlockSpec((BM, MODEL_DIM // 2),
                               lambda i, wexp, ps, cnt: (i, 0)),
    )
    return pl.pallas_call(
        _gemm_body,
        grid_spec=grid_spec,
        out_shape=jax.ShapeDtypeStruct((NPAD, MODEL_DIM // 2), jnp.int32),
        interpret=_INTERPRET,
    )(wexp, pad_start, counts, x_sorted, u, d)


@functools.partial(jax.jit, static_argnames=())
def kernel(hidden_states, gate_w, gate_up_w, down_w):
    orig_shape = hidden_states.shape
    x = hidden_states.reshape(-1, orig_shape[-1])
    u = gate_up_w[:, :, INTER_DIM:].astype(jnp.bfloat16)
    d = down_w.astype(jnp.bfloat16)

    e1, e2, xb, h1, h2 = _routing(x, gate_w.T)

    x_sorted, inv, wexp, pad_start, counts = _dispatch_sc(
        e1.reshape(-1), e2.reshape(-1),
        xb, h1.reshape(NW // 2, 16), h2.reshape(NW // 2, 16))
    rows = _grouped_gemm(x_sorted, u, d, wexp, pad_start, counts)
    out = _combine_sc(rows, inv)
    return out.reshape(orig_shape)


# speedup vs baseline: 1.0983x; 1.0166x over previous
import functools

import jax
import jax.numpy as jnp
from jax import lax
from jax.experimental import pallas as pl
from jax.experimental.pallas import tpu as pltpu

NUM_EXPERTS = 8
TOP_K = 2
MODEL_DIM = 768
INTER_DIM = 768
T = 2048
N = T * TOP_K
BM = 256
NPAD = N + NUM_EXPERTS * BM
NWIN = NPAD // BM


def _routing_body(x_ref, gwt_ref, e1_ref, e2_ref, xb_ref, h1_ref, h2_ref):
    x = x_ref[...]
    logits = lax.dot_general(
        x, gwt_ref[...], (((1,), (0,)), ((), ())),
        preferred_element_type=jnp.float32)
    idx = lax.broadcasted_iota(jnp.int32, logits.shape, 1)
    r1 = jnp.max(logits, axis=1, keepdims=True)
    i1 = jnp.min(jnp.where(logits == r1, idx, NUM_EXPERTS),
                 axis=1, keepdims=True)
    l2 = jnp.where(idx == i1, -jnp.inf, logits)
    r2 = jnp.max(l2, axis=1, keepdims=True)
    i2 = jnp.min(jnp.where(l2 == r2, idx, NUM_EXPERTS),
                 axis=1, keepdims=True)
    e1_ref[...] = i1
    e2_ref[...] = i2
    lane16 = lax.broadcasted_iota(jnp.int32, (i1.shape[0], 16), 1)
    oh1 = (lane16 == i1).astype(jnp.int32)
    oh2 = (lane16 == i2).astype(jnp.int32)
    hb = i1.shape[0] // 2
    h1_ref[0, 0:1, :] = jnp.sum(oh1[:hb], axis=0, keepdims=True)
    h1_ref[0, 1:2, :] = jnp.sum(oh1[hb:], axis=0, keepdims=True)
    h2_ref[0, 0:1, :] = jnp.sum(oh2[:hb], axis=0, keepdims=True)
    h2_ref[0, 1:2, :] = jnp.sum(oh2[hb:], axis=0, keepdims=True)
    half = MODEL_DIM // 2
    lo = pltpu.bitcast(x[:, :half].astype(jnp.bfloat16).astype(jnp.float32),
                       jnp.int32)
    hi = pltpu.bitcast(x[:, half:].astype(jnp.bfloat16).astype(jnp.float32),
                       jnp.int32)
    xb_ref[...] = hi | lax.shift_right_logical(lo, 16)


def _routing(x, gwt):
    bt = 256
    return pl.pallas_call(
        _routing_body,
        grid=(T // bt,),
        in_specs=[
            pl.BlockSpec((bt, MODEL_DIM), lambda i: (i, 0)),
            pl.BlockSpec((MODEL_DIM, NUM_EXPERTS), lambda i: (0, 0)),
        ],
        out_specs=[
            pl.BlockSpec((bt, 1), lambda i: (i, 0)),
            pl.BlockSpec((bt, 1), lambda i: (i, 0)),
            pl.BlockSpec((bt, MODEL_DIM // 2), lambda i: (i, 0)),
            pl.BlockSpec((1, 2, 16), lambda i: (i, 0, 0)),
            pl.BlockSpec((1, 2, 16), lambda i: (i, 0, 0)),
        ],
        out_shape=[
            jax.ShapeDtypeStruct((T, 1), jnp.int32),
            jax.ShapeDtypeStruct((T, 1), jnp.int32),
            jax.ShapeDtypeStruct((T, MODEL_DIM // 2), jnp.int32),
            jax.ShapeDtypeStruct((T // 256, 2, 16), jnp.int32),
            jax.ShapeDtypeStruct((T // 256, 2, 16), jnp.int32),
        ],
    )(x, gwt)


NC = 2
NS = 16
LANES = 16
NW = NC * NS
SLOTS_W = N // NW
CHUNK = 32
NCHUNK = SLOTS_W // CHUNK
TOK_W = T // NW
NWEXP = 48


def _sc_mesh():
    from jax.experimental.pallas import tpu_sc as plsc
    return plsc.VectorSubcoreMesh(core_axis_name="c", subcore_axis_name="s")


def _dispatch_body(e1_hbm, e2_hbm, x_hbm, h1_hbm, h2_hbm,
                   xs_hbm, inv_hbm, wexp_hbm, ps_hbm, cnt_hbm,
                   eids_v, hist_v, pos2d, inv_v, meta_v, wexp_v,
                   bufa, bufb, sema, semb):
    from jax.experimental.pallas import tpu_sc as plsc
    wid = lax.axis_index("c") * NS + lax.axis_index("s")
    base = wid * SLOTS_W
    lane = lax.iota(jnp.int32, LANES)

    @pl.when(wid < NW // 2)
    def _():
        pltpu.sync_copy(e1_hbm.at[pl.ds(wid * SLOTS_W, SLOTS_W)], eids_v)

    @pl.when(wid >= NW // 2)
    def _():
        pltpu.sync_copy(e2_hbm.at[pl.ds((wid - NW // 2) * SLOTS_W, SLOTS_W)],
                        eids_v)
    pltpu.sync_copy(h1_hbm, hist_v.at[pl.ds(0, NW // 2)])
    pltpu.sync_copy(h2_hbm, hist_v.at[pl.ds(NW // 2, NW // 2)])

    tot = jnp.zeros((LANES,), jnp.int32)
    pre = jnp.zeros((LANES,), jnp.int32)
    for r in range(NW):
        row = hist_v[r, :]
        tot = tot + row
        pre = pre + jnp.where(r < wid, row, 0)

    pe = jnp.bitwise_and(tot + (BM - 1), -BM)
    incl = plsc.cumsum(pe)
    ps = incl - pe
    myb = ps + pre

    @pl.when(wid == 0)
    def _():
        meta_v[...] = ps
        pltpu.sync_copy(meta_v, ps_hbm)
        meta_v[...] = tot
        pltpu.sync_copy(meta_v, cnt_hbm)
        for k in range(NWEXP // LANES):
            wstart = (lane + k * LANES) * BM
            acc = jnp.zeros((LANES,), jnp.int32)
            for e in range(NUM_EXPERTS):
                incl_e = jnp.sum(jnp.where(lane == e, incl, 0))
                acc = acc + (wstart >= incl_e).astype(jnp.int32)
            wexp_v[pl.ds(k * LANES, LANES)] = jnp.minimum(acc, NUM_EXPERTS - 1)
        pltpu.sync_copy(wexp_v, wexp_hbm)

    for k in range(SLOTS_W // LANES):
        v = eids_v[pl.ds(k * LANES, LANES)]
        pos = jnp.zeros((LANES,), jnp.int32)
        for e in range(NUM_EXPERTS):
            m = v == e
            mi = m.astype(jnp.int32)
            r = plsc.cumsum(mi)
            base_e = jnp.sum(jnp.where(lane == e, myb, 0))
            pos = jnp.where(m, base_e + r - 1, pos)
            cnt_e = jnp.sum(mi)
            myb = myb + jnp.where(lane == e, cnt_e, 0)
        pos2d[k // 2, pl.ds((k % 2) * LANES, LANES)] = pos
        inv_v[pl.ds(k * LANES, LANES)] = pos
    pltpu.sync_copy(inv_v, inv_hbm.at[pl.ds(base, SLOTS_W)])

    handles = [None] * NCHUNK
    for c in range(NCHUNK):
        buf, sem = (bufa, sema) if c % 2 == 0 else (bufb, semb)
        if c >= 2:
            handles[c - 2].wait()
        tokbase = lax.rem(base + c * CHUNK, T)
        pltpu.sync_copy(x_hbm.at[pl.ds(tokbase, CHUNK)], buf)
        handles[c] = pltpu.async_copy(buf, xs_hbm.at[pos2d.at[c]], sem)
    handles[NCHUNK - 2].wait()
    handles[NCHUNK - 1].wait()


def _dispatch_sc(e1, e2, x, h1, h2):
    f = functools.partial(
        pl.kernel,
        out_type=[
            jax.ShapeDtypeStruct((NPAD, MODEL_DIM // 2), jnp.int32),
            jax.ShapeDtypeStruct((N,), jnp.int32),
            jax.ShapeDtypeStruct((NWEXP,), jnp.int32),
            jax.ShapeDtypeStruct((LANES,), jnp.int32),
            jax.ShapeDtypeStruct((LANES,), jnp.int32),
        ],
        mesh=_sc_mesh(),
        compiler_params=pltpu.CompilerParams(needs_layout_passes=False),
        scratch_types=[
            pltpu.VMEM((SLOTS_W,), jnp.int32),
            pltpu.VMEM((NW, 16), jnp.int32),
            pltpu.VMEM((NCHUNK, CHUNK), jnp.int32),
            pltpu.VMEM((SLOTS_W,), jnp.int32),
            pltpu.VMEM((LANES,), jnp.int32),
            pltpu.VMEM((NWEXP,), jnp.int32),
            pltpu.VMEM((CHUNK, MODEL_DIM // 2), jnp.int32),
            pltpu.VMEM((CHUNK, MODEL_DIM // 2), jnp.int32),
            pltpu.SemaphoreType.DMA,
            pltpu.SemaphoreType.DMA,
        ],
    )(_dispatch_body)
    return f(e1, e2, x, h1, h2)


def _combine_body(rows_hbm, inv_hbm, out_hbm, idx_v, r1a, r2a, r1b, r2b,
                  obfa, obfb, s1, s2, s3, s4):
    from jax.experimental.pallas import tpu_sc as plsc
    wid = lax.axis_index("c") * NS + lax.axis_index("s")
    tb = wid * TOK_W
    half = MODEL_DIM // 2
    lane = lax.iota(jnp.int32, LANES)
    pltpu.sync_copy(inv_hbm.at[pl.ds(tb, CHUNK)], idx_v.at[0])
    pltpu.sync_copy(inv_hbm.at[pl.ds(tb + CHUNK, CHUNK)], idx_v.at[1])
    pltpu.sync_copy(inv_hbm.at[pl.ds(T + tb, CHUNK)], idx_v.at[2])
    pltpu.sync_copy(inv_hbm.at[pl.ds(T + tb + CHUNK, CHUNK)], idx_v.at[3])
    groups_row = half // LANES
    gh = [None] * 4
    gh[0] = pltpu.async_copy(rows_hbm.at[idx_v.at[0]], r1a, s1)
    gh[1] = pltpu.async_copy(rows_hbm.at[idx_v.at[2]], r2a, s2)
    gh[2] = pltpu.async_copy(rows_hbm.at[idx_v.at[1]], r1b, s3)
    gh[3] = pltpu.async_copy(rows_hbm.at[idx_v.at[3]], r2b, s4)
    wh = [None] * 2
    for c in range(TOK_W // CHUNK):
        r1, r2, obf = (r1a, r2a, obfa) if c == 0 else (r1b, r2b, obfb)
        gh[2 * c].wait()
        gh[2 * c + 1].wait()

        def add_row(i, _):
            off = i * MODEL_DIM
            for j in range(groups_row):
                sl = pl.ds(j * LANES, LANES)
                w1 = r1[i, sl]
                w2 = r2[i, sl]
                losum = (plsc.bitcast(w1 << 16, jnp.float32)
                         + plsc.bitcast(w2 << 16, jnp.float32))
                hisum = (plsc.bitcast(w1 & -65536, jnp.float32)
                         + plsc.bitcast(w2 & -65536, jnp.float32))
                obf[pl.ds(off + j * LANES, LANES)] = losum
                obf[pl.ds(off + half + j * LANES, LANES)] = hisum
            return 0

        lax.fori_loop(0, CHUNK, add_row, 0)
        wh[c] = pltpu.async_copy(
            obf, out_hbm.at[pl.ds((tb + c * CHUNK) * MODEL_DIM,
                                  CHUNK * MODEL_DIM)], s1 if c == 0 else s2)
    wh[0].wait()
    wh[1].wait()


def _combine_sc(rows, inv):
    f = functools.partial(
        pl.kernel,
        out_type=jax.ShapeDtypeStruct((T * MODEL_DIM,), jnp.float32),
        mesh=_sc_mesh(),
        compiler_params=pltpu.CompilerParams(needs_layout_passes=False),
        scratch_types=[
            pltpu.VMEM((4, CHUNK), jnp.int32),
            pltpu.VMEM((CHUNK, MODEL_DIM // 2), jnp.int32),
            pltpu.VMEM((CHUNK, MODEL_DIM // 2), jnp.int32),
            pltpu.VMEM((CHUNK, MODEL_DIM // 2), jnp.int32),
            pltpu.VMEM((CHUNK, MODEL_DIM // 2), jnp.int32),
            pltpu.VMEM((CHUNK * MODEL_DIM,), jnp.float32),
            pltpu.VMEM((CHUNK * MODEL_DIM,), jnp.float32),
            pltpu.SemaphoreType.DMA,
            pltpu.SemaphoreType.DMA,
            pltpu.SemaphoreType.DMA,
            pltpu.SemaphoreType.DMA,
        ],
    )(_combine_body)
    return f(rows, inv)


def _dispatch_jnp(x, eids):
    counts = jnp.bincount(eids, length=NUM_EXPERTS)
    padded = (counts + BM - 1) // BM * BM
    pad_start = jnp.concatenate([jnp.zeros((1,), jnp.int32),
                                 jnp.cumsum(padded)[:-1].astype(jnp.int32)])
    cs_excl = jnp.concatenate([jnp.zeros((1,), jnp.int32),
                               jnp.cumsum(counts)[:-1].astype(jnp.int32)])
    order = jnp.argsort(eids, stable=True)
    eids_sorted = eids[order]
    rank = jnp.arange(N, dtype=jnp.int32) - cs_excl[eids_sorted]
    pos_sorted = pad_start[eids_sorted] + rank
    inv = jnp.zeros((N,), jnp.int32).at[order].set(pos_sorted)
    tok = jnp.arange(N, dtype=jnp.int32) % T
    x_sorted = jnp.zeros((NPAD, MODEL_DIM), x.dtype).at[inv].set(x[tok])
    pad_incl = pad_start + padded
    wstarts = jnp.arange(NWIN, dtype=jnp.int32) * BM
    wexp = jnp.minimum(
        jnp.sum(wstarts[:, None] >= pad_incl[None, :], axis=1),
        NUM_EXPERTS - 1).astype(jnp.int32)
    return x_sorted, inv, wexp, pad_start.astype(jnp.int32), counts.astype(jnp.int32)


def _gemm_body(wexp_ref, ps_ref, cnt_ref, xs_ref, u_ref, d_ref, out_ref):
    w = pl.program_id(0)
    e = wexp_ref[w]
    loc = (lax.broadcasted_iota(jnp.int32, (BM, 1), 0)
           + w * BM - ps_ref[e])
    active = loc < cnt_ref[e]

    @pl.when(w * BM - ps_ref[e] < cnt_ref[e])
    def _():
        _gemm_compute(active, xs_ref, u_ref, d_ref, out_ref)


def _gemm_compute(active, xs_ref, u_ref, d_ref, out_ref):
    packed = jnp.where(active, xs_ref[...], 0)
    lo = pltpu.bitcast(lax.shift_left(packed, 16), jnp.float32)
    hi = pltpu.bitcast(packed & -65536, jnp.float32)
    x = jnp.concatenate([lo, hi], axis=1).astype(jnp.bfloat16)
    u = lax.dot_general(x, u_ref[0], (((1,), (0,)), ((), ())),
                        preferred_element_type=jnp.float32)
    h = (u * u).astype(jnp.bfloat16)
    acc = lax.dot_general(h, d_ref[0], (((1,), (0,)), ((), ())),
                          preferred_element_type=jnp.float32)
    half = MODEL_DIM // 2
    alo = pltpu.bitcast(acc[:, :half].astype(jnp.bfloat16).astype(jnp.float32),
                        jnp.int32)
    ahi = pltpu.bitcast(acc[:, half:].astype(jnp.bfloat16).astype(jnp.float32),
                        jnp.int32)
    out_ref[...] = ahi | lax.shift_right_logical(alo, 16)


def _grouped_gemm(x_sorted, u, d, wexp, pad_start, counts):
    grid_spec = pltpu.PrefetchScalarGridSpec(
        num_scalar_prefetch=3,
        grid=(NWIN,),
        in_specs=[
            pl.BlockSpec((BM, MODEL_DIM // 2), lambda i, wexp, ps, cnt: (i, 0)),
            pl.BlockSpec((1, MODEL_DIM, INTER_DIM),
                         lambda i, wexp, ps, cnt: (wexp[i], 0, 0)),
            pl.BlockSpec((1, INTER_DIM, MODEL_DIM),
                         lambda i, wexp, ps, cnt: (wexp[i], 0, 0)),
        ],
        out_specs=pl.BlockSpec((BM, MODEL_DIM // 2),
                               lambda i, wexp, ps, cnt: (i, 0)),
    )
    return pl.pallas_call(
        _gemm_body,
        grid_spec=grid_spec,
        out_shape=jax.ShapeDtypeStruct((NPAD, MODEL_DIM // 2), jnp.int32),
    )(wexp, pad_start, counts, x_sorted, u, d)


@functools.partial(jax.jit, static_argnames=())
def kernel(hidden_states, gate_w, gate_up_w, down_w):
    orig_shape = hidden_states.shape
    x = hidden_states.reshape(-1, orig_shape[-1])
    u = gate_up_w[:, :, INTER_DIM:].astype(jnp.bfloat16)
    d = down_w.astype(jnp.bfloat16)

    e1, e2, xb, h1, h2 = _routing(x, gate_w.T)

    x_sorted, inv, wexp, pad_start, counts = _dispatch_sc(
        e1.reshape(-1), e2.reshape(-1),
        xb, h1.reshape(NW // 2, 16), h2.reshape(NW // 2, 16))
    rows = _grouped_gemm(x_sorted, u, d, wexp, pad_start, counts)
    out = _combine_sc(rows, inv)
    return out.reshape(orig_shape)
